# baseline (speedup 1.0000x reference)
import functools

import jax
import jax.numpy as jnp
from jax import lax
from jax.experimental import pallas as pl
from jax.experimental.pallas import tpu as pltpu
from jax.experimental.pallas import tpu_sc as plsc

_LANES = 16


@functools.lru_cache(maxsize=None)
def _build_sc_add(B, S, D, dtype):
    mesh = plsc.VectorSubcoreMesh(core_axis_name="c", subcore_axis_name="s")
    NC, NS = mesh.num_cores, mesh.num_subcores
    NW = NC * NS
    SPW = S // NW
    R = 32
    NT = SPW // R
    CH = D // _LANES

    @functools.partial(
        pl.kernel,
        out_type=jax.ShapeDtypeStruct((B, S, D), dtype),
        mesh=mesh,
        scratch_types=[
            pltpu.VMEM((R, D), dtype),
            pltpu.VMEM((R, D), dtype),
        ],
    )
    def k(x_hbm, pos_hbm, out_hbm, pos_v, x_v):
        wid = lax.axis_index("s") * NC + lax.axis_index("c")
        p0 = wid * SPW

        def tile(t, carry):
            row0 = p0 + t * R
            pltpu.sync_copy(pos_hbm.at[pl.ds(row0, R)], pos_v)
            for b in range(B):
                pltpu.sync_copy(x_hbm.at[b, pl.ds(row0, R)], x_v)

                def add(i, c2):
                    r = i // CH
                    col = (i % CH) * _LANES
                    x_v[r, pl.ds(col, _LANES)] = (
                        x_v[r, pl.ds(col, _LANES)] + pos_v[r, pl.ds(col, _LANES)]
                    )
                    return c2

                lax.fori_loop(0, R * CH, add, 0)
                pltpu.sync_copy(x_v, out_hbm.at[b, pl.ds(row0, R)])
            return carry

        lax.fori_loop(0, NT, tile, 0)

    return k


def kernel(x, pos_table):
    B, S, D = x.shape
    return _build_sc_add(B, S, D, x.dtype)(x, pos_table)


# speedup vs baseline: 1.1606x; 1.1606x over previous
import functools

import jax
import jax.numpy as jnp
from jax import lax
from jax.experimental import pallas as pl
from jax.experimental.pallas import tpu as pltpu
from jax.experimental.pallas import tpu_sc as plsc

_LANES = 16


@functools.lru_cache(maxsize=None)
def _build_sc_add(B, S, D, dtype):
    mesh = plsc.VectorSubcoreMesh(core_axis_name="c", subcore_axis_name="s")
    NC, NS = mesh.num_cores, mesh.num_subcores
    NW = NC * NS
    SPW = S // NW
    R = 16
    NT = SPW // R
    W = R * D

    scratch = (
        [pltpu.VMEM((W,), dtype) for _ in range(2)]
        + [pltpu.VMEM((W,), dtype) for _ in range(2 * B)]
        + [pltpu.SemaphoreType.DMA for _ in range(4)]
    )

    @functools.partial(
        pl.kernel,
        out_type=jax.ShapeDtypeStruct((B, S * D), dtype),
        mesh=mesh,
        scratch_types=scratch,
    )
    def k(x_hbm, pos_hbm, out_hbm, *scr):
        pbuf = [scr[0], scr[1]]
        xbuf = [[scr[2 + b] for b in range(B)], [scr[2 + B + b] for b in range(B)]]
        in_sem = [scr[2 + 2 * B], scr[3 + 2 * B]]
        out_sem = [scr[4 + 2 * B], scr[5 + 2 * B]]

        wid = lax.axis_index("s") * NC + lax.axis_index("c")
        base = wid * (SPW * D)

        def issue_in(t):
            sl = t % 2
            off = base + t * W
            descs = [pltpu.async_copy(pos_hbm.at[pl.ds(off, W)], pbuf[sl], in_sem[sl])]
            for b in range(B):
                descs.append(
                    pltpu.async_copy(x_hbm.at[b, pl.ds(off, W)], xbuf[sl][b], in_sem[sl])
                )
            return descs

        def issue_out(t):
            sl = t % 2
            off = base + t * W
            return [
                pltpu.async_copy(xbuf[sl][b], out_hbm.at[b, pl.ds(off, W)], out_sem[sl])
                for b in range(B)
            ]

        def compute(sl):
            pv = pbuf[sl]
            xb = xbuf[sl]

            @plsc.parallel_loop(0, W // _LANES, 1, unroll=8)
            def _(i):
                off = pl.multiple_of(i * _LANES, _LANES)
                p = pv[pl.ds(off, _LANES)]
                for b in range(B):
                    xb[b][pl.ds(off, _LANES)] = xb[b][pl.ds(off, _LANES)] + p

        outs = {}
        cur_in = issue_in(0)
        for t in range(NT):
            for d in cur_in:
                d.wait()
            compute(t % 2)
            outs[t] = issue_out(t)
            if t + 1 < NT:
                if t - 1 in outs:
                    for d in outs.pop(t - 1):
                        d.wait()
                cur_in = issue_in(t + 1)
        for t in sorted(outs):
            for d in outs[t]:
                d.wait()

    return k


def kernel(x, pos_table):
    B, S, D = x.shape
    x2 = x.reshape(B, S * D)
    pos2 = pos_table.reshape(-1)
    out2 = _build_sc_add(B, S, D, x.dtype)(x2, pos2)
    return out2.reshape(B, S, D)


# speedup vs baseline: 3.2723x; 2.8195x over previous
import functools

import jax
import jax.numpy as jnp
from jax import lax
from jax.experimental import pallas as pl
from jax.experimental.pallas import tpu as pltpu
from jax.experimental.pallas import tpu_sc as plsc

_LANES = 16


@functools.lru_cache(maxsize=None)
def _build_sc_add(B, S, D, dtype):
    mesh = plsc.VectorSubcoreMesh(core_axis_name="c", subcore_axis_name="s")
    NC, NS = mesh.num_cores, mesh.num_subcores
    NW = NC * NS
    SPW = S // NW
    R = 8
    NT = SPW // R
    CH = D // _LANES

    scratch = (
        [pltpu.VMEM((R, D), dtype) for _ in range(2)]
        + [pltpu.VMEM((R, D), dtype) for _ in range(2 * B)]
        + [pltpu.VMEM((R, D), dtype) for _ in range(2 * B)]
        + [pltpu.SemaphoreType.DMA for _ in range(4)]
    )

    @functools.partial(
        pl.kernel,
        out_type=jax.ShapeDtypeStruct((B, S, D), dtype),
        mesh=mesh,
        scratch_types=scratch,
    )
    def k(x_hbm, pos_hbm, out_hbm, *scr):
        pbuf = [scr[0], scr[1]]
        xbuf = [[scr[2 + b] for b in range(B)], [scr[2 + B + b] for b in range(B)]]
        obuf = [[scr[2 + 2 * B + b] for b in range(B)],
                [scr[2 + 3 * B + b] for b in range(B)]]
        in_sem = [scr[2 + 4 * B], scr[3 + 4 * B]]
        out_sem = [scr[4 + 4 * B], scr[5 + 4 * B]]

        wid = lax.axis_index("s") * NC + lax.axis_index("c")
        p0 = wid * SPW

        ins, outs = {}, {}

        def issue_in(t):
            sl = t % 2
            row0 = p0 + t * R
            descs = [pltpu.async_copy(pos_hbm.at[pl.ds(row0, R)], pbuf[sl], in_sem[sl])]
            for b in range(B):
                descs.append(
                    pltpu.async_copy(x_hbm.at[b, pl.ds(row0, R)], xbuf[sl][b], in_sem[sl])
                )
            ins[t] = descs

        def issue_out(t):
            sl = t % 2
            row0 = p0 + t * R
            outs[t] = [
                pltpu.async_copy(obuf[sl][b], out_hbm.at[b, pl.ds(row0, R)], out_sem[sl])
                for b in range(B)
            ]

        def compute(t):
            sl = t % 2
            pv = pbuf[sl]
            xb = xbuf[sl]
            ob = obuf[sl]

            @plsc.parallel_loop(0, R * CH, 1, unroll=4)
            def _(i):
                r = i // CH
                col = (i % CH) * _LANES
                p = pv[r, pl.ds(col, _LANES)]
                for b in range(B):
                    ob[b][r, pl.ds(col, _LANES)] = xb[b][r, pl.ds(col, _LANES)] + p

        issue_in(0)
        issue_in(1)
        for t in range(NT):
            for d in ins.pop(t):
                d.wait()
            if t >= 2:
                for d in outs.pop(t - 2):
                    d.wait()
            compute(t)
            issue_out(t)
            if t + 2 < NT:
                issue_in(t + 2)
        for t in sorted(outs):
            for d in outs[t]:
                d.wait()

    return k


def kernel(x, pos_table):
    B, S, D = x.shape
    return _build_sc_add(B, S, D, x.dtype)(x, pos_table)
